# Initial kernel scaffold; baseline (speedup 1.0000x reference)
import jax
import jax.numpy as jnp
from jax.experimental import pallas as pl


def kernel(x, edge_index, batch, W1_0, b1_0, W2_0, b2_0, W1_1, b1_1, W2_1, b2_1, W1_2, b1_2, W2_2, b2_2):
    raise NotImplementedError("write your pallas kernel here")


# speedup vs baseline: 6.5406x; 6.5406x over previous
import functools

import jax
import jax.numpy as jnp
from jax import lax
from jax.experimental import pallas as pl
from jax.experimental.pallas import tpu as pltpu
from jax.experimental.pallas import tpu_sc as plsc

N = 10000
E = 320000
DIM = 128
G = 128

NC = 2
NS = 16
NW = NC * NS
EDGES_PER_TILE = E // NW
CHUNK = 80
NCHUNK = EDGES_PER_TILE // CHUNK
ROWS_PER_TILE = 624
ROWS_TAIL = N - NS * ROWS_PER_TILE

ROW_BLK = 1000
NBLK = N // ROW_BLK


def _sc_aggregate(h, src_r, dst_r):
    mesh = plsc.VectorSubcoreMesh(core_axis_name="c", subcore_axis_name="s")

    @functools.partial(
        pl.kernel,
        out_type=jax.ShapeDtypeStruct((NC, N, DIM), jnp.float32),
        mesh=mesh,
        scratch_types=[
            pltpu.VMEM((NCHUNK, CHUNK), jnp.int32),
            pltpu.VMEM((NCHUNK, CHUNK), jnp.int32),
            pltpu.VMEM((CHUNK, DIM), jnp.float32),
            pltpu.VMEM_SHARED((N, DIM), jnp.float32),
            pltpu.SemaphoreType.DMA,
        ],
    )
    def agg_kernel(h_hbm, src_hbm, dst_hbm, out_hbm, src_v, dst_v, gbuf, acc, sem):
        cid = lax.axis_index("c")
        sid = lax.axis_index("s")
        wid = sid * NC + cid

        pltpu.sync_copy(src_hbm.at[wid], src_v)
        pltpu.sync_copy(dst_hbm.at[wid], dst_v)
        row0 = sid * ROWS_PER_TILE
        pltpu.sync_copy(h_hbm.at[pl.ds(row0, ROWS_PER_TILE)],
                        acc.at[pl.ds(row0, ROWS_PER_TILE)])

        @pl.when(sid == NS - 1)
        def _():
            pltpu.sync_copy(h_hbm.at[pl.ds(NS * ROWS_PER_TILE, ROWS_TAIL)],
                            acc.at[pl.ds(NS * ROWS_PER_TILE, ROWS_TAIL)])

        plsc.subcore_barrier()

        def body(j, carry):
            pltpu.async_copy(h_hbm.at[src_v.at[j]], gbuf, sem).wait()
            pltpu.sync_copy(gbuf, acc.at[dst_v.at[j]], add=True)
            return carry

        lax.fori_loop(0, NCHUNK, body, 0)
        plsc.subcore_barrier()
        pltpu.sync_copy(acc.at[pl.ds(row0, ROWS_PER_TILE)],
                        out_hbm.at[cid, pl.ds(row0, ROWS_PER_TILE)])

        @pl.when(sid == NS - 1)
        def _():
            pltpu.sync_copy(acc.at[pl.ds(NS * ROWS_PER_TILE, ROWS_TAIL)],
                            out_hbm.at[cid, pl.ds(NS * ROWS_PER_TILE, ROWS_TAIL)])

    return agg_kernel(h, src_r, dst_r)


def _tc_mlp_pool(p, h, batch_col, W1, b1, W2, b2):
    def body(p_ref, h_ref, batch_ref, w1_ref, b1_ref, w2_ref, b2_ref,
             hout_ref, pool_ref, acc_ref, cnt_ref):
        i = pl.program_id(0)
        z = p_ref[0] + p_ref[1] - h_ref[...]
        a = jnp.maximum(
            jnp.dot(z, w1_ref[...], preferred_element_type=jnp.float32)
            + b1_ref[...], 0.0)
        hn = jnp.maximum(
            jnp.dot(a, w2_ref[...], preferred_element_type=jnp.float32)
            + b2_ref[...], 0.0)
        hout_ref[...] = hn

        onehot = (batch_ref[...] ==
                  lax.broadcasted_iota(jnp.int32, (ROW_BLK, G), 1)
                  ).astype(jnp.float32)
        pT = lax.dot_general(onehot, hn, (((0,), (0,)), ((), ())),
                             preferred_element_type=jnp.float32)
        c = lax.dot_general(onehot, jnp.ones((ROW_BLK, 1), jnp.float32),
                            (((0,), (0,)), ((), ())),
                            preferred_element_type=jnp.float32)

        @pl.when(i == 0)
        def _():
            acc_ref[...] = pT
            cnt_ref[...] = c

        @pl.when(i > 0)
        def _():
            acc_ref[...] += pT
            cnt_ref[...] += c

        @pl.when(i == NBLK - 1)
        def _():
            pool_ref[...] = acc_ref[...] / jnp.maximum(cnt_ref[...], 1.0)

    return pl.pallas_call(
        body,
        grid=(NBLK,),
        in_specs=[
            pl.BlockSpec((NC, ROW_BLK, DIM), lambda i: (0, i, 0)),
            pl.BlockSpec((ROW_BLK, DIM), lambda i: (i, 0)),
            pl.BlockSpec((ROW_BLK, 1), lambda i: (i, 0)),
            pl.BlockSpec((DIM, DIM), lambda i: (0, 0)),
            pl.BlockSpec((1, DIM), lambda i: (0, 0)),
            pl.BlockSpec((DIM, DIM), lambda i: (0, 0)),
            pl.BlockSpec((1, DIM), lambda i: (0, 0)),
        ],
        out_specs=[
            pl.BlockSpec((ROW_BLK, DIM), lambda i: (i, 0)),
            pl.BlockSpec((G, DIM), lambda i: (0, 0)),
        ],
        out_shape=[
            jax.ShapeDtypeStruct((N, DIM), jnp.float32),
            jax.ShapeDtypeStruct((G, DIM), jnp.float32),
        ],
        scratch_shapes=[
            pltpu.VMEM((G, DIM), jnp.float32),
            pltpu.VMEM((G, 1), jnp.float32),
        ],
    )(p, h, batch_col, W1, b1, W2, b2)


def kernel(x, edge_index, batch, W1_0, b1_0, W2_0, b2_0, W1_1, b1_1, W2_1,
           b2_1, W1_2, b1_2, W2_2, b2_2):
    src_r = edge_index[0].reshape(NW, NCHUNK, CHUNK)
    dst_r = edge_index[1].reshape(NW, NCHUNK, CHUNK)
    batch_col = batch.reshape(N, 1)
    params = [(W1_0, b1_0.reshape(1, DIM), W2_0, b2_0.reshape(1, DIM)),
              (W1_1, b1_1.reshape(1, DIM), W2_1, b2_1.reshape(1, DIM)),
              (W1_2, b1_2.reshape(1, DIM), W2_2, b2_2.reshape(1, DIM))]

    h = x
    xs = []
    pools = []
    for (W1, b1, W2, b2) in params:
        p = _sc_aggregate(h, src_r, dst_r)
        h, pool = _tc_mlp_pool(p, h, batch_col, W1, b1, W2, b2)
        xs.append(h)
        pools.append(pool)
    return (jnp.concatenate(pools, axis=1), jnp.concatenate(xs, axis=1))


# speedup vs baseline: 10.4210x; 1.5933x over previous
import functools

import jax
import jax.numpy as jnp
from jax import lax
from jax.experimental import pallas as pl
from jax.experimental.pallas import tpu as pltpu
from jax.experimental.pallas import tpu_sc as plsc

N = 10000
E = 320000
DIM = 128
G = 128

NC = 2
NS = 16
NW = NC * NS
EDGES_PER_TILE = E // NW
CHUNK = 80
NCHUNK = EDGES_PER_TILE // CHUNK
ROWS_PER_TILE = 624
ROWS_TAIL = N - NS * ROWS_PER_TILE

ROW_BLK = 1000
NBLK = N // ROW_BLK


def _sc_aggregate(h, src_r, dst_r):
    mesh = plsc.VectorSubcoreMesh(core_axis_name="c", subcore_axis_name="s")

    @functools.partial(
        pl.kernel,
        out_type=jax.ShapeDtypeStruct((NC, N, DIM), jnp.float32),
        mesh=mesh,
        scratch_types=[
            pltpu.VMEM((EDGES_PER_TILE,), jnp.int32),
            pltpu.VMEM((NCHUNK, CHUNK), jnp.int32),
            pltpu.VMEM((2, CHUNK, DIM), jnp.float32),
            pltpu.VMEM_SHARED((N, DIM), jnp.float32),
            pltpu.SemaphoreType.DMA,
            pltpu.SemaphoreType.DMA,
        ],
    )
    def agg_kernel(h_hbm, src_hbm, dst_hbm, out_hbm, src_v, dst_v, gbuf, acc,
                   sem0, sem1):
        cid = lax.axis_index("c")
        sid = lax.axis_index("s")
        wid = sid * NC + cid

        pltpu.sync_copy(src_hbm.at[wid], src_v)
        pltpu.sync_copy(dst_hbm.at[wid], dst_v)
        row0 = sid * ROWS_PER_TILE
        pltpu.sync_copy(h_hbm.at[pl.ds(row0, ROWS_PER_TILE)],
                        acc.at[pl.ds(row0, ROWS_PER_TILE)])

        @pl.when(sid == NS - 1)
        def _():
            pltpu.sync_copy(h_hbm.at[pl.ds(NS * ROWS_PER_TILE, ROWS_TAIL)],
                            acc.at[pl.ds(NS * ROWS_PER_TILE, ROWS_TAIL)])

        plsc.subcore_barrier()

        def src_at(j):
            return src_v.at[pl.ds(j * CHUNK, CHUNK)]

        pltpu.async_copy(h_hbm.at[src_at(0)], gbuf.at[0], sem0)

        def body(i, carry):
            j = 2 * i
            pltpu.async_copy(h_hbm.at[src_at(j + 1)], gbuf.at[1], sem1)
            pltpu.make_async_copy(h_hbm.at[src_at(j)], gbuf.at[0],
                                  sem0).wait()
            pltpu.sync_copy(gbuf.at[0], acc.at[dst_v.at[j]], add=True)
            pltpu.async_copy(h_hbm.at[src_at(j + 2)], gbuf.at[0], sem0)
            pltpu.make_async_copy(h_hbm.at[src_at(j + 1)], gbuf.at[1],
                                  sem1).wait()
            pltpu.sync_copy(gbuf.at[1], acc.at[dst_v.at[j + 1]], add=True)
            return carry

        lax.fori_loop(0, (NCHUNK - 1) // 2, body, 0)
        pltpu.make_async_copy(h_hbm.at[src_at(NCHUNK - 1)], gbuf.at[0],
                              sem0).wait()
        pltpu.sync_copy(gbuf.at[0], acc.at[dst_v.at[NCHUNK - 1]], add=True)
        plsc.subcore_barrier()
        pltpu.sync_copy(acc.at[pl.ds(row0, ROWS_PER_TILE)],
                        out_hbm.at[cid, pl.ds(row0, ROWS_PER_TILE)])

        @pl.when(sid == NS - 1)
        def _():
            pltpu.sync_copy(acc.at[pl.ds(NS * ROWS_PER_TILE, ROWS_TAIL)],
                            out_hbm.at[cid, pl.ds(NS * ROWS_PER_TILE, ROWS_TAIL)])

    return agg_kernel(h, src_r, dst_r)


def _tc_mlp_pool(p, h, batch_col, W1, b1, W2, b2):
    def body(p_ref, h_ref, batch_ref, w1_ref, b1_ref, w2_ref, b2_ref,
             hout_ref, pool_ref, acc_ref, cnt_ref):
        i = pl.program_id(0)
        z = p_ref[0] + p_ref[1] - h_ref[...]
        a = jnp.maximum(
            jnp.dot(z, w1_ref[...], preferred_element_type=jnp.float32)
            + b1_ref[...], 0.0)
        hn = jnp.maximum(
            jnp.dot(a, w2_ref[...], preferred_element_type=jnp.float32)
            + b2_ref[...], 0.0)
        hout_ref[...] = hn

        onehot = (batch_ref[...] ==
                  lax.broadcasted_iota(jnp.int32, (ROW_BLK, G), 1)
                  ).astype(jnp.float32)
        pT = lax.dot_general(onehot, hn, (((0,), (0,)), ((), ())),
                             preferred_element_type=jnp.float32)
        c = lax.dot_general(onehot, jnp.ones((ROW_BLK, 1), jnp.float32),
                            (((0,), (0,)), ((), ())),
                            preferred_element_type=jnp.float32)

        @pl.when(i == 0)
        def _():
            acc_ref[...] = pT
            cnt_ref[...] = c

        @pl.when(i > 0)
        def _():
            acc_ref[...] += pT
            cnt_ref[...] += c

        @pl.when(i == NBLK - 1)
        def _():
            pool_ref[...] = acc_ref[...] / jnp.maximum(cnt_ref[...], 1.0)

    return pl.pallas_call(
        body,
        grid=(NBLK,),
        in_specs=[
            pl.BlockSpec((NC, ROW_BLK, DIM), lambda i: (0, i, 0)),
            pl.BlockSpec((ROW_BLK, DIM), lambda i: (i, 0)),
            pl.BlockSpec((ROW_BLK, 1), lambda i: (i, 0)),
            pl.BlockSpec((DIM, DIM), lambda i: (0, 0)),
            pl.BlockSpec((1, DIM), lambda i: (0, 0)),
            pl.BlockSpec((DIM, DIM), lambda i: (0, 0)),
            pl.BlockSpec((1, DIM), lambda i: (0, 0)),
        ],
        out_specs=[
            pl.BlockSpec((ROW_BLK, DIM), lambda i: (i, 0)),
            pl.BlockSpec((G, DIM), lambda i: (0, 0)),
        ],
        out_shape=[
            jax.ShapeDtypeStruct((N, DIM), jnp.float32),
            jax.ShapeDtypeStruct((G, DIM), jnp.float32),
        ],
        scratch_shapes=[
            pltpu.VMEM((G, DIM), jnp.float32),
            pltpu.VMEM((G, 1), jnp.float32),
        ],
    )(p, h, batch_col, W1, b1, W2, b2)


def kernel(x, edge_index, batch, W1_0, b1_0, W2_0, b2_0, W1_1, b1_1, W2_1,
           b2_1, W1_2, b1_2, W2_2, b2_2):
    src_r = edge_index[0].reshape(NW, NCHUNK * CHUNK)
    dst_r = edge_index[1].reshape(NW, NCHUNK, CHUNK)
    batch_col = batch.reshape(N, 1)
    params = [(W1_0, b1_0.reshape(1, DIM), W2_0, b2_0.reshape(1, DIM)),
              (W1_1, b1_1.reshape(1, DIM), W2_1, b2_1.reshape(1, DIM)),
              (W1_2, b1_2.reshape(1, DIM), W2_2, b2_2.reshape(1, DIM))]

    h = x
    xs = []
    pools = []
    for (W1, b1, W2, b2) in params:
        p = _sc_aggregate(h, src_r, dst_r)
        h, pool = _tc_mlp_pool(p, h, batch_col, W1, b1, W2, b2)
        xs.append(h)
        pools.append(pool)
    return (jnp.concatenate(pools, axis=1), jnp.concatenate(xs, axis=1))


# speedup vs baseline: 10.7985x; 1.0362x over previous
import functools

import jax
import jax.numpy as jnp
from jax import lax
from jax.experimental import pallas as pl
from jax.experimental.pallas import tpu as pltpu
from jax.experimental.pallas import tpu_sc as plsc

N = 10000
E = 320000
DIM = 128
G = 128
L_LAYERS = 3

NC = 2
NS = 16
NW = NC * NS
EDGES_PER_TILE = E // NW
CHUNK = 80
NCHUNK = EDGES_PER_TILE // CHUNK
ROWS_PER_TILE = 624
ROWS_TAIL = N - NS * ROWS_PER_TILE

ROW_BLK = 2000
NBLK = N // ROW_BLK


def _sc_aggregate(h, src_r, dst_r):
    mesh = plsc.VectorSubcoreMesh(core_axis_name="c", subcore_axis_name="s")

    @functools.partial(
        pl.kernel,
        out_type=jax.ShapeDtypeStruct((NC, N, DIM), jnp.float32),
        mesh=mesh,
        scratch_types=[
            pltpu.VMEM((EDGES_PER_TILE,), jnp.int32),
            pltpu.VMEM((NCHUNK, CHUNK), jnp.int32),
            pltpu.VMEM((2, CHUNK, DIM), jnp.float32),
            pltpu.VMEM_SHARED((N, DIM), jnp.float32),
            pltpu.SemaphoreType.DMA,
            pltpu.SemaphoreType.DMA,
        ],
    )
    def agg_kernel(h_hbm, src_hbm, dst_hbm, out_hbm, src_v, dst_v, gbuf, acc,
                   sem0, sem1):
        cid = lax.axis_index("c")
        sid = lax.axis_index("s")
        wid = sid * NC + cid

        pltpu.sync_copy(src_hbm.at[wid], src_v)
        pltpu.sync_copy(dst_hbm.at[wid], dst_v)
        row0 = sid * ROWS_PER_TILE
        pltpu.sync_copy(h_hbm.at[pl.ds(row0, ROWS_PER_TILE)],
                        acc.at[pl.ds(row0, ROWS_PER_TILE)])

        @pl.when(sid == NS - 1)
        def _():
            pltpu.sync_copy(h_hbm.at[pl.ds(NS * ROWS_PER_TILE, ROWS_TAIL)],
                            acc.at[pl.ds(NS * ROWS_PER_TILE, ROWS_TAIL)])

        plsc.subcore_barrier()

        def src_at(j):
            return src_v.at[pl.ds(j * CHUNK, CHUNK)]

        pltpu.async_copy(h_hbm.at[src_at(0)], gbuf.at[0], sem0)

        def body(i, carry):
            j = 2 * i
            pltpu.async_copy(h_hbm.at[src_at(j + 1)], gbuf.at[1], sem1)
            pltpu.make_async_copy(h_hbm.at[src_at(j)], gbuf.at[0],
                                  sem0).wait()
            pltpu.sync_copy(gbuf.at[0], acc.at[dst_v.at[j]], add=True)
            pltpu.async_copy(h_hbm.at[src_at(j + 2)], gbuf.at[0], sem0)
            pltpu.make_async_copy(h_hbm.at[src_at(j + 1)], gbuf.at[1],
                                  sem1).wait()
            pltpu.sync_copy(gbuf.at[1], acc.at[dst_v.at[j + 1]], add=True)
            return carry

        lax.fori_loop(0, (NCHUNK - 1) // 2, body, 0)
        pltpu.make_async_copy(h_hbm.at[src_at(NCHUNK - 1)], gbuf.at[0],
                              sem0).wait()
        pltpu.sync_copy(gbuf.at[0], acc.at[dst_v.at[NCHUNK - 1]], add=True)
        plsc.subcore_barrier()
        pltpu.sync_copy(acc.at[pl.ds(row0, ROWS_PER_TILE)],
                        out_hbm.at[cid, pl.ds(row0, ROWS_PER_TILE)])

        @pl.when(sid == NS - 1)
        def _():
            pltpu.sync_copy(acc.at[pl.ds(NS * ROWS_PER_TILE, ROWS_TAIL)],
                            out_hbm.at[cid, pl.ds(NS * ROWS_PER_TILE, ROWS_TAIL)])

    return agg_kernel(h, src_r, dst_r)


def _tc_mlp_pool(p, h, batch_col, W1, b1, W2, b2, layer, xcat=None):
    last = layer == L_LAYERS - 1

    def body(*refs):
        if last:
            (p_ref, h_ref, batch_ref, w1_ref, b1_ref, w2_ref, b2_ref,
             _xin_ref, xcat_ref, pool_ref, acc_ref, cnt_ref) = refs
            hout_ref = None
        elif layer == 0:
            (p_ref, h_ref, batch_ref, w1_ref, b1_ref, w2_ref, b2_ref,
             hout_ref, xcat_ref, pool_ref, acc_ref, cnt_ref) = refs
        else:
            (p_ref, h_ref, batch_ref, w1_ref, b1_ref, w2_ref, b2_ref,
             _xin_ref, hout_ref, xcat_ref, pool_ref, acc_ref, cnt_ref) = refs
        i = pl.program_id(0)
        z = p_ref[0] + p_ref[1] - h_ref[...]
        a = jnp.maximum(
            jnp.dot(z, w1_ref[...], preferred_element_type=jnp.float32)
            + b1_ref[...], 0.0)
        hn = jnp.maximum(
            jnp.dot(a, w2_ref[...], preferred_element_type=jnp.float32)
            + b2_ref[...], 0.0)
        if hout_ref is not None:
            hout_ref[...] = hn
        xcat_ref[...] = hn

        onehot = (batch_ref[...] ==
                  lax.broadcasted_iota(jnp.int32, (ROW_BLK, G), 1)
                  ).astype(jnp.float32)
        pT = lax.dot_general(onehot, hn, (((0,), (0,)), ((), ())),
                             preferred_element_type=jnp.float32)
        c = lax.dot_general(onehot, jnp.ones((ROW_BLK, 1), jnp.float32),
                            (((0,), (0,)), ((), ())),
                            preferred_element_type=jnp.float32)

        @pl.when(i == 0)
        def _():
            acc_ref[...] = pT
            cnt_ref[...] = c

        @pl.when(i > 0)
        def _():
            acc_ref[...] += pT
            cnt_ref[...] += c

        @pl.when(i == NBLK - 1)
        def _():
            pool_ref[...] = acc_ref[...] / jnp.maximum(cnt_ref[...], 1.0)

    in_specs = [
        pl.BlockSpec((NC, ROW_BLK, DIM), lambda i: (0, i, 0)),
        pl.BlockSpec((ROW_BLK, DIM), lambda i: (i, 0)),
        pl.BlockSpec((ROW_BLK, 1), lambda i: (i, 0)),
        pl.BlockSpec((DIM, DIM), lambda i: (0, 0)),
        pl.BlockSpec((1, DIM), lambda i: (0, 0)),
        pl.BlockSpec((DIM, DIM), lambda i: (0, 0)),
        pl.BlockSpec((1, DIM), lambda i: (0, 0)),
    ]
    args = [p, h, batch_col, W1, b1, W2, b2]
    xcat_spec = pl.BlockSpec((ROW_BLK, DIM), lambda i, _l=layer: (i, _l))
    hout_spec = pl.BlockSpec((ROW_BLK, DIM), lambda i: (i, 0))
    pool_spec = pl.BlockSpec((G, DIM), lambda i: (0, 0))
    xcat_shape = jax.ShapeDtypeStruct((N, L_LAYERS * DIM), jnp.float32)
    hout_shape = jax.ShapeDtypeStruct((N, DIM), jnp.float32)
    pool_shape = jax.ShapeDtypeStruct((G, DIM), jnp.float32)
    io_aliases = {}
    if layer == 0:
        out_specs = [hout_spec, xcat_spec, pool_spec]
        out_shape = [hout_shape, xcat_shape, pool_shape]
    else:
        in_specs.append(pl.BlockSpec(memory_space=pl.ANY))
        args.append(xcat)
        if last:
            out_specs = [xcat_spec, pool_spec]
            out_shape = [xcat_shape, pool_shape]
            io_aliases = {7: 0}
        else:
            out_specs = [hout_spec, xcat_spec, pool_spec]
            out_shape = [hout_shape, xcat_shape, pool_shape]
            io_aliases = {7: 1}

    out = pl.pallas_call(
        body,
        grid=(NBLK,),
        in_specs=in_specs,
        out_specs=out_specs,
        out_shape=out_shape,
        scratch_shapes=[
            pltpu.VMEM((G, DIM), jnp.float32),
            pltpu.VMEM((G, 1), jnp.float32),
        ],
        input_output_aliases=io_aliases,
    )(*args)
    if last:
        return None, out[0], out[1]
    return out[0], out[1], out[2]


def kernel(x, edge_index, batch, W1_0, b1_0, W2_0, b2_0, W1_1, b1_1, W2_1,
           b2_1, W1_2, b1_2, W2_2, b2_2):
    src_r = edge_index[0].reshape(NW, NCHUNK * CHUNK)
    dst_r = edge_index[1].reshape(NW, NCHUNK, CHUNK)
    batch_col = batch.reshape(N, 1)
    params = [(W1_0, b1_0.reshape(1, DIM), W2_0, b2_0.reshape(1, DIM)),
              (W1_1, b1_1.reshape(1, DIM), W2_1, b2_1.reshape(1, DIM)),
              (W1_2, b1_2.reshape(1, DIM), W2_2, b2_2.reshape(1, DIM))]

    h = x
    xcat = None
    pools = []
    for layer, (W1, b1, W2, b2) in enumerate(params):
        p = _sc_aggregate(h, src_r, dst_r)
        h, xcat, pool = _tc_mlp_pool(p, h, batch_col, W1, b1, W2, b2,
                                     layer, xcat)
        pools.append(pool)
    return (jnp.concatenate(pools, axis=1), xcat)


# speedup vs baseline: 10.9592x; 1.0149x over previous
import functools

import jax
import jax.numpy as jnp
from jax import lax
from jax.experimental import pallas as pl
from jax.experimental.pallas import tpu as pltpu
from jax.experimental.pallas import tpu_sc as plsc

N = 10000
E = 320000
DIM = 128
G = 128
L_LAYERS = 3

NC = 2
NS = 16
NW = NC * NS
EDGES_PER_TILE = E // NW
CHUNK = 80
NCHUNK = EDGES_PER_TILE // CHUNK
ROWS_PER_TILE = 624
ROWS_TAIL = N - NS * ROWS_PER_TILE

ROW_BLK = 2000
NBLK = N // ROW_BLK


def _sc_aggregate(h, src_r, dst_r):
    mesh = plsc.VectorSubcoreMesh(core_axis_name="c", subcore_axis_name="s")

    @functools.partial(
        pl.kernel,
        out_type=jax.ShapeDtypeStruct((NC, N, DIM), jnp.float32),
        mesh=mesh,
        scratch_types=[
            pltpu.VMEM((EDGES_PER_TILE,), jnp.int32),
            pltpu.VMEM((NCHUNK, CHUNK), jnp.int32),
            pltpu.VMEM((2, CHUNK, DIM), jnp.float32),
            pltpu.VMEM_SHARED((N, DIM), jnp.float32),
            pltpu.SemaphoreType.DMA,
            pltpu.SemaphoreType.DMA,
            pltpu.SemaphoreType.DMA,
        ],
    )
    def agg_kernel(h_hbm, src_hbm, dst_hbm, out_hbm, src_v, dst_v, gbuf, acc,
                   sem0, sem1, sem2):
        cid = lax.axis_index("c")
        sid = lax.axis_index("s")
        wid = sid * NC + cid

        def src_at(j):
            return src_v.at[pl.ds(j * CHUNK, CHUNK)]

        pltpu.async_copy(src_hbm.at[wid], src_v, sem0)
        pltpu.async_copy(dst_hbm.at[wid], dst_v, sem1)
        row0 = sid * ROWS_PER_TILE
        pltpu.async_copy(h_hbm.at[pl.ds(row0, ROWS_PER_TILE)],
                         acc.at[pl.ds(row0, ROWS_PER_TILE)], sem2)

        @pl.when(sid == NS - 1)
        def _():
            pltpu.sync_copy(h_hbm.at[pl.ds(NS * ROWS_PER_TILE, ROWS_TAIL)],
                            acc.at[pl.ds(NS * ROWS_PER_TILE, ROWS_TAIL)])

        pltpu.make_async_copy(src_hbm.at[wid], src_v, sem0).wait()
        pltpu.async_copy(h_hbm.at[src_at(0)], gbuf.at[0], sem0)
        pltpu.make_async_copy(dst_hbm.at[wid], dst_v, sem1).wait()
        pltpu.make_async_copy(
            h_hbm.at[pl.ds(row0, ROWS_PER_TILE)],
            acc.at[pl.ds(row0, ROWS_PER_TILE)], sem2).wait()
        plsc.subcore_barrier()


        def body(i, carry):
            j = 2 * i
            pltpu.async_copy(h_hbm.at[src_at(j + 1)], gbuf.at[1], sem1)
            pltpu.make_async_copy(h_hbm.at[src_at(j)], gbuf.at[0],
                                  sem0).wait()
            pltpu.sync_copy(gbuf.at[0], acc.at[dst_v.at[j]], add=True)
            pltpu.async_copy(h_hbm.at[src_at(j + 2)], gbuf.at[0], sem0)
            pltpu.make_async_copy(h_hbm.at[src_at(j + 1)], gbuf.at[1],
                                  sem1).wait()
            pltpu.sync_copy(gbuf.at[1], acc.at[dst_v.at[j + 1]], add=True)
            return carry

        lax.fori_loop(0, (NCHUNK - 1) // 2, body, 0)
        pltpu.make_async_copy(h_hbm.at[src_at(NCHUNK - 1)], gbuf.at[0],
                              sem0).wait()
        pltpu.sync_copy(gbuf.at[0], acc.at[dst_v.at[NCHUNK - 1]], add=True)
        plsc.subcore_barrier()
        pltpu.sync_copy(acc.at[pl.ds(row0, ROWS_PER_TILE)],
                        out_hbm.at[cid, pl.ds(row0, ROWS_PER_TILE)])

        @pl.when(sid == NS - 1)
        def _():
            pltpu.sync_copy(acc.at[pl.ds(NS * ROWS_PER_TILE, ROWS_TAIL)],
                            out_hbm.at[cid, pl.ds(NS * ROWS_PER_TILE, ROWS_TAIL)])

    return agg_kernel(h, src_r, dst_r)


def _tc_mlp_pool(p, h, batch_col, W1, b1, W2, b2, layer, xcat=None):
    last = layer == L_LAYERS - 1

    def body(*refs):
        if last:
            (p_ref, h_ref, batch_ref, w1_ref, b1_ref, w2_ref, b2_ref,
             _xin_ref, xcat_ref, pool_ref, acc_ref, cnt_ref) = refs
            hout_ref = None
        elif layer == 0:
            (p_ref, h_ref, batch_ref, w1_ref, b1_ref, w2_ref, b2_ref,
             hout_ref, xcat_ref, pool_ref, acc_ref, cnt_ref) = refs
        else:
            (p_ref, h_ref, batch_ref, w1_ref, b1_ref, w2_ref, b2_ref,
             _xin_ref, hout_ref, xcat_ref, pool_ref, acc_ref, cnt_ref) = refs
        i = pl.program_id(0)
        z = p_ref[0] + p_ref[1] - h_ref[...]
        a = jnp.maximum(
            jnp.dot(z, w1_ref[...], preferred_element_type=jnp.float32)
            + b1_ref[...], 0.0)
        hn = jnp.maximum(
            jnp.dot(a, w2_ref[...], preferred_element_type=jnp.float32)
            + b2_ref[...], 0.0)
        if hout_ref is not None:
            hout_ref[...] = hn
        xcat_ref[...] = hn

        onehot = (batch_ref[...] ==
                  lax.broadcasted_iota(jnp.int32, (ROW_BLK, G), 1)
                  ).astype(jnp.float32)
        pT = lax.dot_general(onehot, hn, (((0,), (0,)), ((), ())),
                             preferred_element_type=jnp.float32)
        c = lax.dot_general(onehot, jnp.ones((ROW_BLK, 1), jnp.float32),
                            (((0,), (0,)), ((), ())),
                            preferred_element_type=jnp.float32)

        @pl.when(i == 0)
        def _():
            acc_ref[...] = pT
            cnt_ref[...] = c

        @pl.when(i > 0)
        def _():
            acc_ref[...] += pT
            cnt_ref[...] += c

        @pl.when(i == NBLK - 1)
        def _():
            pool_ref[...] = acc_ref[...] / jnp.maximum(cnt_ref[...], 1.0)

    in_specs = [
        pl.BlockSpec((NC, ROW_BLK, DIM), lambda i: (0, i, 0)),
        pl.BlockSpec((ROW_BLK, DIM), lambda i: (i, 0)),
        pl.BlockSpec((ROW_BLK, 1), lambda i: (i, 0)),
        pl.BlockSpec((DIM, DIM), lambda i: (0, 0)),
        pl.BlockSpec((1, DIM), lambda i: (0, 0)),
        pl.BlockSpec((DIM, DIM), lambda i: (0, 0)),
        pl.BlockSpec((1, DIM), lambda i: (0, 0)),
    ]
    args = [p, h, batch_col, W1, b1, W2, b2]
    xcat_spec = pl.BlockSpec((ROW_BLK, DIM), lambda i, _l=layer: (i, _l))
    hout_spec = pl.BlockSpec((ROW_BLK, DIM), lambda i: (i, 0))
    pool_spec = pl.BlockSpec((G, DIM), lambda i: (0, 0))
    xcat_shape = jax.ShapeDtypeStruct((N, L_LAYERS * DIM), jnp.float32)
    hout_shape = jax.ShapeDtypeStruct((N, DIM), jnp.float32)
    pool_shape = jax.ShapeDtypeStruct((G, DIM), jnp.float32)
    io_aliases = {}
    if layer == 0:
        out_specs = [hout_spec, xcat_spec, pool_spec]
        out_shape = [hout_shape, xcat_shape, pool_shape]
    else:
        in_specs.append(pl.BlockSpec(memory_space=pl.ANY))
        args.append(xcat)
        if last:
            out_specs = [xcat_spec, pool_spec]
            out_shape = [xcat_shape, pool_shape]
            io_aliases = {7: 0}
        else:
            out_specs = [hout_spec, xcat_spec, pool_spec]
            out_shape = [hout_shape, xcat_shape, pool_shape]
            io_aliases = {7: 1}

    out = pl.pallas_call(
        body,
        grid=(NBLK,),
        in_specs=in_specs,
        out_specs=out_specs,
        out_shape=out_shape,
        scratch_shapes=[
            pltpu.VMEM((G, DIM), jnp.float32),
            pltpu.VMEM((G, 1), jnp.float32),
        ],
        input_output_aliases=io_aliases,
    )(*args)
    if last:
        return None, out[0], out[1]
    return out[0], out[1], out[2]


def kernel(x, edge_index, batch, W1_0, b1_0, W2_0, b2_0, W1_1, b1_1, W2_1,
           b2_1, W1_2, b1_2, W2_2, b2_2):
    src_r = edge_index[0].reshape(NW, NCHUNK * CHUNK)
    dst_r = edge_index[1].reshape(NW, NCHUNK, CHUNK)
    batch_col = batch.reshape(N, 1)
    params = [(W1_0, b1_0.reshape(1, DIM), W2_0, b2_0.reshape(1, DIM)),
              (W1_1, b1_1.reshape(1, DIM), W2_1, b2_1.reshape(1, DIM)),
              (W1_2, b1_2.reshape(1, DIM), W2_2, b2_2.reshape(1, DIM))]

    h = x
    xcat = None
    pools = []
    for layer, (W1, b1, W2, b2) in enumerate(params):
        p = _sc_aggregate(h, src_r, dst_r)
        h, xcat, pool = _tc_mlp_pool(p, h, batch_col, W1, b1, W2, b2,
                                     layer, xcat)
        pools.append(pool)
    return (jnp.concatenate(pools, axis=1), xcat)
